# Initial kernel scaffold; baseline (speedup 1.0000x reference)
import jax
import jax.numpy as jnp
from jax.experimental import pallas as pl


def kernel(features, nodes, to_neighs_list, self_feats):
    raise NotImplementedError("write your pallas kernel here")


# speedup vs baseline: 3.4332x; 3.4332x over previous
import functools

import jax
import jax.numpy as jnp
from jax import lax
from jax.experimental import pallas as pl
from jax.experimental.pallas import tpu as pltpu
from jax.experimental.pallas import tpu_sc as plsc

N_NODES = 10000
D = 512
B = 2048
K = 32
L = 16
NW = 32
RPW = B // NW
R = 4
NCH = RPW // R
CPD = D // L


def _sc_body(feat_hbm, ids_hbm, self_hbm, out_hbm,
             ids_v, rows_v, self_v, out_v, w_v, sem):
    cid = lax.axis_index("c")
    sid = lax.axis_index("s")
    wid = sid * 2 + cid
    row0 = wid * RPW

    pltpu.sync_copy(ids_hbm.at[pl.ds(row0 * K, RPW * K)], ids_v)

    pos_a = lax.iota(jnp.int32, 16)
    pos_b = pos_a + 16

    def chunk_body(ch, carry):
        r0 = ch * R
        idx_ref = ids_v.at[pl.ds(r0 * K, R * K)]
        pltpu.async_copy(feat_hbm.at[idx_ref], rows_v, sem).wait()
        pltpu.sync_copy(self_hbm.at[pl.ds(row0 + r0, R)], self_v)

        for rr in range(R):
            base = (r0 + rr) * K
            a = ids_v[pl.ds(base, L)]
            b = ids_v[pl.ds(base + L, L)]

            def dd_body(s, dup):
                dup_a, dup_b = dup
                ia = jnp.maximum(pos_a - s, 0) + base
                ib = jnp.maximum(pos_b - s, 0) + base
                pa = plsc.load_gather(ids_v, [ia])
                pb = plsc.load_gather(ids_v, [ib])
                ca = jnp.where((a == pa) & (pos_a >= s), 1, 0)
                cb = jnp.where((b == pb) & (pos_b >= s), 1, 0)
                return dup_a | ca, dup_b | cb

            z16 = jnp.zeros((L,), jnp.int32)
            dup_a, dup_b = lax.fori_loop(1, K, dd_body, (z16, z16))

            wa = jnp.where(dup_a != 0, 0.0, 1.0).astype(jnp.float32)
            wb = jnp.where(dup_b != 0, 0.0, 1.0).astype(jnp.float32)
            n_unique = jnp.sum(wa) + jnp.sum(wb)
            inv = jnp.full((L,), 1.0, jnp.float32) / (
                n_unique + jnp.zeros((L,), jnp.float32))
            w_v[pl.ds(0, L)] = wa * inv
            w_v[pl.ds(L, L)] = wb * inv

            def acc_body(j, acc):
                wj = plsc.load_gather(w_v, [jnp.zeros((L,), jnp.int32) + j])
                rbase = rr * K + j
                return tuple(acc[c] + rows_v[rbase, pl.ds(c * L, L)] * wj
                             for c in range(CPD))

            acc0 = tuple(jnp.zeros((L,), jnp.float32) for _ in range(CPD))
            acc = lax.fori_loop(0, K, acc_body, acc0)

            for c in range(CPD):
                aggc = acc[c]
                sf = self_v[rr, pl.ds(c * L, L)]
                out_v[rr, pl.ds(c * L, L)] = sf - aggc
                out_v[rr, pl.ds(D + c * L, L)] = aggc

        pltpu.sync_copy(out_v, out_hbm.at[pl.ds(row0 + r0, R)])
        return carry

    lax.fori_loop(0, NCH, chunk_body, 0)


@jax.jit
def _intra_agg(features, ids_flat, self_feats):
    mesh = plsc.VectorSubcoreMesh(core_axis_name="c", subcore_axis_name="s")
    f = functools.partial(
        pl.kernel,
        mesh=mesh,
        compiler_params=pltpu.CompilerParams(needs_layout_passes=False),
        out_type=jax.ShapeDtypeStruct((B, 2 * D), jnp.float32),
        scratch_types=[
            pltpu.VMEM((RPW * K,), jnp.int32),
            pltpu.VMEM((R * K, D), jnp.float32),
            pltpu.VMEM((R, D), jnp.float32),
            pltpu.VMEM((R, 2 * D), jnp.float32),
            pltpu.VMEM((K,), jnp.float32),
            pltpu.SemaphoreType.DMA,
        ],
    )(_sc_body)
    return f(features, ids_flat, self_feats)


def kernel(features, nodes, to_neighs_list, self_feats):
    del nodes
    ids_flat = to_neighs_list.astype(jnp.int32).reshape(-1)
    return _intra_agg(features, ids_flat, self_feats)


# speedup vs baseline: 5.8719x; 1.7103x over previous
import functools

import jax
import jax.numpy as jnp
from jax import lax
from jax.experimental import pallas as pl
from jax.experimental.pallas import tpu as pltpu
from jax.experimental.pallas import tpu_sc as plsc

N_NODES = 10000
D = 512
B = 2048
K = 32
L = 16
NW = 32
RPW = B // NW
R = 2
NCH = RPW // R
NPAIR = NCH // 2
CPD = D // L


def _sc_body(feat_hbm, ids_hbm, self_hbm, out_hbm,
             ids_v, table_v, nd_v, inv_v,
             rows0, rows1, self0, self1, out0, out1,
             sem_r0, sem_r1, sem_s0, sem_s1):
    cid = lax.axis_index("c")
    sid = lax.axis_index("s")
    wid = sid * 2 + cid
    row0 = wid * RPW

    pltpu.sync_copy(ids_hbm.at[pl.ds(row0 * K, RPW * K)], ids_v)

    pos_a = lax.iota(jnp.int32, L)
    pos_b = pos_a + L
    zf = jnp.zeros((L,), jnp.float32)
    zi = jnp.zeros((L,), jnp.int32)

    def dd_body(r, carry):
        base = r * K
        a = ids_v[pl.ds(base, L)]
        b = ids_v[pl.ds(base + L, L)]
        plsc.store_scatter(table_v, [a], pos_a)
        plsc.store_scatter(table_v, [b], pos_b)
        ga = plsc.load_gather(table_v, [a])
        gb = plsc.load_gather(table_v, [b])
        fa = ga == pos_a
        fb = gb == pos_b
        id0 = plsc.load_gather(ids_v, [zi + base])
        ids_v[pl.ds(base, L)] = jnp.where(fa, a, id0)
        ids_v[pl.ds(base + L, L)] = jnp.where(fb, b, id0)
        fa_f = jnp.where(fa, 1.0, 0.0).astype(jnp.float32)
        fb_f = jnp.where(fb, 1.0, 0.0).astype(jnp.float32)
        n_unique = jnp.sum(fa_f) + jnp.sum(fb_f)
        nd_v[r, pl.ds(0, L)] = (K - n_unique) + zf
        inv_v[r, pl.ds(0, L)] = (1.0 + zf) / (n_unique + zf)
        return carry

    lax.fori_loop(0, RPW, dd_body, 0)

    def rows_dma(ch, buf, sem):
        idx = ids_v.at[pl.ds(ch * R * K, R * K)]
        return pltpu.make_async_copy(feat_hbm.at[idx], buf, sem)

    def self_dma(ch, buf, sem):
        return pltpu.make_async_copy(
            self_hbm.at[pl.ds(row0 + ch * R, R)], buf, sem)

    def compute(ch, rows_b, self_b, out_b):
        for rr in range(R):
            def acc_body(j, acc):
                rbase = rr * K + j
                return tuple(acc[c] + rows_b[rbase, pl.ds(c * L, L)]
                             for c in range(CPD))

            acc0 = tuple(jnp.zeros((L,), jnp.float32) for _ in range(CPD))
            acc = lax.fori_loop(0, K, acc_body, acc0)

            r = ch * R + rr
            nd = nd_v[r, pl.ds(0, L)]
            inv = inv_v[r, pl.ds(0, L)]
            for c in range(CPD):
                r0c = rows_b[rr * K, pl.ds(c * L, L)]
                aggc = (acc[c] - nd * r0c) * inv
                out_b[rr, pl.ds(c * L, L)] = self_b[rr, pl.ds(c * L, L)] - aggc
                out_b[rr, pl.ds(D + c * L, L)] = aggc

    rows_dma(0, rows0, sem_r0).start()
    self_dma(0, self0, sem_s0).start()

    def pair_body(i2, carry):
        ch0 = i2 * 2
        ch1 = ch0 + 1
        rows_dma(ch1, rows1, sem_r1).start()
        self_dma(ch1, self1, sem_s1).start()

        rows_dma(ch0, rows0, sem_r0).wait()
        self_dma(ch0, self0, sem_s0).wait()
        compute(ch0, rows0, self0, out0)
        pltpu.sync_copy(out0, out_hbm.at[pl.ds(row0 + ch0 * R, R)])

        @pl.when(i2 < NPAIR - 1)
        def _():
            rows_dma(ch0 + 2, rows0, sem_r0).start()
            self_dma(ch0 + 2, self0, sem_s0).start()

        rows_dma(ch1, rows1, sem_r1).wait()
        self_dma(ch1, self1, sem_s1).wait()
        compute(ch1, rows1, self1, out1)
        pltpu.sync_copy(out1, out_hbm.at[pl.ds(row0 + ch1 * R, R)])
        return carry

    lax.fori_loop(0, NPAIR, pair_body, 0)


@jax.jit
def _intra_agg(features, ids_flat, self_feats):
    mesh = plsc.VectorSubcoreMesh(core_axis_name="c", subcore_axis_name="s")
    f = functools.partial(
        pl.kernel,
        mesh=mesh,
        compiler_params=pltpu.CompilerParams(needs_layout_passes=False),
        out_type=jax.ShapeDtypeStruct((B, 2 * D), jnp.float32),
        scratch_types=[
            pltpu.VMEM((RPW * K,), jnp.int32),
            pltpu.VMEM((N_NODES,), jnp.int32),
            pltpu.VMEM((RPW, L), jnp.float32),
            pltpu.VMEM((RPW, L), jnp.float32),
            pltpu.VMEM((R * K, D), jnp.float32),
            pltpu.VMEM((R * K, D), jnp.float32),
            pltpu.VMEM((R, D), jnp.float32),
            pltpu.VMEM((R, D), jnp.float32),
            pltpu.VMEM((R, 2 * D), jnp.float32),
            pltpu.VMEM((R, 2 * D), jnp.float32),
            pltpu.SemaphoreType.DMA,
            pltpu.SemaphoreType.DMA,
            pltpu.SemaphoreType.DMA,
            pltpu.SemaphoreType.DMA,
        ],
    )(_sc_body)
    return f(features, ids_flat, self_feats)


def kernel(features, nodes, to_neighs_list, self_feats):
    del nodes
    ids_flat = to_neighs_list.astype(jnp.int32).reshape(-1)
    return _intra_agg(features, ids_flat, self_feats)
